# Initial kernel scaffold; baseline (speedup 1.0000x reference)
import jax
import jax.numpy as jnp
from jax.experimental import pallas as pl


def kernel(node_features, edge_features, params, senders, receivers):
    raise NotImplementedError("write your pallas kernel here")


# speedup vs baseline: 3.4203x; 3.4203x over previous
import functools

import jax
import jax.numpy as jnp
from jax import lax
from jax.experimental import pallas as pl
from jax.experimental.pallas import tpu as pltpu
from jax.experimental.pallas import tpu_sc as plsc

_N = 10000
_E = 320000
_D = 128
_NC, _NS = 2, 16
_NW = _NC * _NS
_EPW = _E // _NW
_CH = 400
_NCH = _EPW // _CH
_CHS = 200
_NCHS = _EPW // _CHS
_RPAD = 10240
_RPT = _RPAD // _NS

_F32 = jnp.float32


def _ln(o, s, b):
    mu = jnp.mean(o, axis=-1, keepdims=True)
    d = o - mu
    var = jnp.mean(d * d, axis=-1, keepdims=True)
    return d * lax.rsqrt(var + 1e-5) * s + b


def _stats_body(x_ref, o_ref):
    x = x_ref[...]
    s = jnp.sum(x, axis=0)
    q = jnp.sum(x * x, axis=0)
    o_ref[...] = jnp.stack([s, q])[None]


def _col_stats(x, block):
    n, dp = x.shape
    grid = n // block
    out = pl.pallas_call(
        _stats_body,
        grid=(grid,),
        in_specs=[pl.BlockSpec((block, dp), lambda i: (i, 0))],
        out_specs=pl.BlockSpec((1, 2, dp), lambda i: (i, 0, 0)),
        out_shape=jax.ShapeDtypeStruct((grid, 2, dp), _F32),
    )(x)
    return jnp.sum(out, axis=0)


def _enc_body(x_ref, w0_ref, b0_ref, w1_ref, b1_ref, s_ref, t_ref, o_ref):
    h = jnp.dot(x_ref[...], w0_ref[...], preferred_element_type=_F32) + b0_ref[...]
    h = jnp.maximum(h, 0.0)
    o = jnp.dot(h, w1_ref[...], preferred_element_type=_F32) + b1_ref[...]
    o_ref[...] = _ln(o, s_ref[...], t_ref[...])


def _enc(x, w0, b0, w1, b1, lns, lnb, block):
    n, dp = x.shape
    grid = n // block
    wspec = [
        pl.BlockSpec((dp, _D), lambda i: (0, 0)),
        pl.BlockSpec((1, _D), lambda i: (0, 0)),
        pl.BlockSpec((_D, _D), lambda i: (0, 0)),
        pl.BlockSpec((1, _D), lambda i: (0, 0)),
        pl.BlockSpec((1, _D), lambda i: (0, 0)),
        pl.BlockSpec((1, _D), lambda i: (0, 0)),
    ]
    return pl.pallas_call(
        _enc_body,
        grid=(grid,),
        in_specs=[pl.BlockSpec((block, dp), lambda i: (i, 0))] + wspec,
        out_specs=pl.BlockSpec((block, _D), lambda i: (i, 0)),
        out_shape=jax.ShapeDtypeStruct((n, _D), _F32),
    )(x, w0, b0, w1, b1, lns, lnb)


def _prep_body(x_ref, ws_ref, wr_ref, ps_ref, pr_ref):
    x = x_ref[...]
    ps_ref[...] = jnp.dot(x, ws_ref[...], preferred_element_type=_F32)
    pr_ref[...] = jnp.dot(x, wr_ref[...], preferred_element_type=_F32)


def _prep(nodes, ws, wr, block=2000):
    grid = _N // block
    return pl.pallas_call(
        _prep_body,
        grid=(grid,),
        in_specs=[pl.BlockSpec((block, _D), lambda i: (i, 0)),
                  pl.BlockSpec((_D, _D), lambda i: (0, 0)),
                  pl.BlockSpec((_D, _D), lambda i: (0, 0))],
        out_specs=[pl.BlockSpec((block, _D), lambda i: (i, 0)),
                   pl.BlockSpec((block, _D), lambda i: (i, 0))],
        out_shape=[jax.ShapeDtypeStruct((_N, _D), _F32),
                   jax.ShapeDtypeStruct((_N, _D), _F32)],
    )(nodes, ws, wr)


def _edge_body(gs_ref, gr_ref, e_ref, we_ref, b0_ref, w1_ref, b1_ref,
               s_ref, t_ref, o_ref):
    e = e_ref[...]
    pre = jnp.dot(e, we_ref[...], preferred_element_type=_F32)
    pre = pre + gs_ref[...] + gr_ref[...] + b0_ref[...]
    h = jnp.maximum(pre, 0.0)
    o = jnp.dot(h, w1_ref[...], preferred_element_type=_F32) + b1_ref[...]
    o_ref[...] = e + _ln(o, s_ref[...], t_ref[...])


def _edge_mlp(gs, gr, edges, we, b0, w1, b1, lns, lnb, block=2000):
    grid = _E // block
    dspec = pl.BlockSpec((block, _D), lambda i: (i, 0))
    wspec = [
        pl.BlockSpec((_D, _D), lambda i: (0, 0)),
        pl.BlockSpec((1, _D), lambda i: (0, 0)),
        pl.BlockSpec((_D, _D), lambda i: (0, 0)),
        pl.BlockSpec((1, _D), lambda i: (0, 0)),
        pl.BlockSpec((1, _D), lambda i: (0, 0)),
        pl.BlockSpec((1, _D), lambda i: (0, 0)),
    ]
    return pl.pallas_call(
        _edge_body,
        grid=(grid,),
        in_specs=[dspec, dspec, dspec] + wspec,
        out_specs=dspec,
        out_shape=jax.ShapeDtypeStruct((_E, _D), _F32),
    )(gs, gr, edges, we, b0, w1, b1, lns, lnb)


def _node_body(x_ref, a0_ref, a1_ref, wn_ref, wa_ref, b0_ref, w1_ref, b1_ref,
               s_ref, t_ref, o_ref):
    x = x_ref[...]
    a = a0_ref[...] + a1_ref[...]
    pre = (jnp.dot(x, wn_ref[...], preferred_element_type=_F32)
           + jnp.dot(a, wa_ref[...], preferred_element_type=_F32)
           + b0_ref[...])
    h = jnp.maximum(pre, 0.0)
    o = jnp.dot(h, w1_ref[...], preferred_element_type=_F32) + b1_ref[...]
    o_ref[...] = x + _ln(o, s_ref[...], t_ref[...])


def _node_mlp(nodes, a0, a1, wn, wa, b0, w1, b1, lns, lnb, block=2000):
    grid = _N // block
    dspec = pl.BlockSpec((block, _D), lambda i: (i, 0))
    wspec = [
        pl.BlockSpec((_D, _D), lambda i: (0, 0)),
        pl.BlockSpec((_D, _D), lambda i: (0, 0)),
        pl.BlockSpec((1, _D), lambda i: (0, 0)),
        pl.BlockSpec((_D, _D), lambda i: (0, 0)),
        pl.BlockSpec((1, _D), lambda i: (0, 0)),
        pl.BlockSpec((1, _D), lambda i: (0, 0)),
        pl.BlockSpec((1, _D), lambda i: (0, 0)),
    ]
    return pl.pallas_call(
        _node_body,
        grid=(grid,),
        in_specs=[dspec, dspec, dspec] + wspec,
        out_specs=dspec,
        out_shape=jax.ShapeDtypeStruct((_N, _D), _F32),
    )(nodes, a0, a1, wn, wa, b0, w1, b1, lns, lnb)


def _dec_body(x_ref, w0_ref, b0_ref, w1_ref, b1_ref, o_ref):
    h = jnp.dot(x_ref[...], w0_ref[...], preferred_element_type=_F32) + b0_ref[...]
    h = jnp.maximum(h, 0.0)
    o_ref[...] = jnp.dot(h, w1_ref[...], preferred_element_type=_F32) + b1_ref[...]


def _dec(nodes, w0, b0, w1p, b1p, block=2000):
    grid = _N // block
    dspec = pl.BlockSpec((block, _D), lambda i: (i, 0))
    wspec = [
        pl.BlockSpec((_D, _D), lambda i: (0, 0)),
        pl.BlockSpec((1, _D), lambda i: (0, 0)),
        pl.BlockSpec((_D, _D), lambda i: (0, 0)),
        pl.BlockSpec((1, _D), lambda i: (0, 0)),
    ]
    return pl.pallas_call(
        _dec_body,
        grid=(grid,),
        in_specs=[dspec] + wspec,
        out_specs=dspec,
        out_shape=jax.ShapeDtypeStruct((_N, _D), _F32),
    )(nodes, w0, b0, w1p, b1p)


def _sc_mesh():
    return plsc.VectorSubcoreMesh(core_axis_name="c", subcore_axis_name="s")


def _gather_call(ps, pr, s_idx, r_idx):
    @functools.partial(
        pl.kernel,
        mesh=_sc_mesh(),
        out_type=[jax.ShapeDtypeStruct((_E, _D), _F32),
                  jax.ShapeDtypeStruct((_E, _D), _F32)],
        scratch_types=[pltpu.VMEM((_CH,), jnp.int32),
                       pltpu.VMEM((_CH, _D), _F32),
                       pltpu.SemaphoreType.DMA],
    )
    def k(ps_hbm, pr_hbm, s_hbm, r_hbm, gs_hbm, gr_hbm, idx_v, rows_v, sem):
        wid = lax.axis_index("s") * _NC + lax.axis_index("c")
        base = wid * _EPW

        def body(i, carry):
            off = base + i * _CH
            pltpu.sync_copy(s_hbm.at[pl.ds(off, _CH)], idx_v)
            pltpu.async_copy(ps_hbm.at[idx_v], rows_v, sem).wait()
            pltpu.sync_copy(rows_v, gs_hbm.at[pl.ds(off, _CH)])
            pltpu.sync_copy(r_hbm.at[pl.ds(off, _CH)], idx_v)
            pltpu.async_copy(pr_hbm.at[idx_v], rows_v, sem).wait()
            pltpu.sync_copy(rows_v, gr_hbm.at[pl.ds(off, _CH)])
            return carry

        lax.fori_loop(0, _NCH, body, 0)

    return k(ps, pr, s_idx, r_idx)


def _scatter_call(edges, r_idx, zacc):
    @functools.partial(
        pl.kernel,
        mesh=_sc_mesh(),
        out_type=jax.ShapeDtypeStruct((2 * _RPAD, _D), _F32),
        scratch_types=[pltpu.VMEM((_CHS,), jnp.int32),
                       pltpu.VMEM((_CHS, _D), _F32),
                       pltpu.VMEM_SHARED((_RPAD, _D), _F32)],
    )
    def k(e_hbm, r_hbm, z_hbm, out_hbm, idx_v, ev, acc_sh):
        c = lax.axis_index("c")
        s = lax.axis_index("s")
        wid = s * _NC + c
        rows0 = s * _RPT
        pltpu.sync_copy(z_hbm.at[pl.ds(rows0, _RPT)], acc_sh.at[pl.ds(rows0, _RPT)])
        plsc.subcore_barrier()
        base = wid * _EPW

        def body(i, carry):
            off = base + i * _CHS
            pltpu.sync_copy(r_hbm.at[pl.ds(off, _CHS)], idx_v)
            pltpu.sync_copy(e_hbm.at[pl.ds(off, _CHS)], ev)
            pltpu.sync_copy(ev, acc_sh.at[idx_v], add=True)
            return carry

        lax.fori_loop(0, _NCHS, body, 0)
        plsc.subcore_barrier()
        pltpu.sync_copy(acc_sh.at[pl.ds(rows0, _RPT)],
                        out_hbm.at[pl.ds(c * _RPAD + rows0, _RPT)])

    return k(edges, r_idx, zacc)


def _fold_norm(sums, count, w0, b0, din):
    s, q = sums[0], sums[1]
    mean = s / count
    std = jnp.sqrt(q / count - mean * mean)
    std = jnp.maximum(std, 1e-8)
    dp = s.shape[0]
    w0p = jnp.pad(w0, ((0, dp - din), (0, 0)))
    w0f = w0p / std[:, None]
    b0f = b0 - (mean / std) @ w0p
    return w0f, b0f.reshape(1, _D)


def _r1(v):
    return v.reshape(1, -1)


def kernel(node_features, edge_features, params, senders, receivers):
    s_idx = senders.astype(jnp.int32)
    r_idx = receivers.astype(jnp.int32)
    nf = jnp.pad(node_features, ((0, 0), (0, 16 - node_features.shape[1])))
    ef = jnp.pad(edge_features, ((0, 0), (0, 8 - edge_features.shape[1])))

    nstats = _col_stats(nf, block=1000)
    estats = _col_stats(ef, block=4000)

    pn = params['node_enc']
    w0f, b0f = _fold_norm(nstats, float(_N), pn['w0'], pn['b0'], node_features.shape[1])
    nodes = _enc(nf, w0f, b0f, pn['w1'], _r1(pn['b1']), _r1(pn['ln_s']),
                 _r1(pn['ln_b']), block=1000)

    pe = params['edge_enc']
    w0f, b0f = _fold_norm(estats, float(_E), pe['w0'], pe['b0'], edge_features.shape[1])
    edges = _enc(ef, w0f, b0f, pe['w1'], _r1(pe['b1']), _r1(pe['ln_s']),
                 _r1(pe['ln_b']), block=4000)

    zacc = jnp.zeros((_RPAD, _D), _F32)

    for blk in params['blocks']:
        be, bn = blk['edge'], blk['node']
        ws, wr, we = be['w0'][:_D], be['w0'][_D:2 * _D], be['w0'][2 * _D:]
        ps, pr = _prep(nodes, ws, wr)
        gs, gr = _gather_call(ps, pr, s_idx, r_idx)
        edges = _edge_mlp(gs, gr, edges, we, _r1(be['b0']), be['w1'],
                          _r1(be['b1']), _r1(be['ln_s']), _r1(be['ln_b']))
        scat = _scatter_call(edges, r_idx, zacc)
        a0 = lax.slice(scat, (0, 0), (_N, _D))
        a1 = lax.slice(scat, (_RPAD, 0), (_RPAD + _N, _D))
        wn, wa = bn['w0'][:_D], bn['w0'][_D:]
        nodes = _node_mlp(nodes, a0, a1, wn, wa, _r1(bn['b0']), bn['w1'],
                          _r1(bn['b1']), _r1(bn['ln_s']), _r1(bn['ln_b']))

    pd = params['decoder']
    w1p = jnp.pad(pd['w1'], ((0, 0), (0, _D - pd['w1'].shape[1])))
    b1p = jnp.pad(pd['b1'], (0, _D - pd['b1'].shape[0]))
    out = _dec(nodes, pd['w0'], _r1(pd['b0']), w1p, _r1(b1p))
    return out[:, :3]


# speedup vs baseline: 3.6481x; 1.0666x over previous
import functools

import jax
import jax.numpy as jnp
from jax import lax
from jax.experimental import pallas as pl
from jax.experimental.pallas import tpu as pltpu
from jax.experimental.pallas import tpu_sc as plsc

_N = 10000
_E = 320000
_D = 128
_NC, _NS = 2, 16
_NW = _NC * _NS
_EPW = _E // _NW
_CH = 400
_NCH = _EPW // _CH
_CHS = 80
_NCHS = _EPW // _CHS
_RPAD = 10240
_RPT = _RPAD // _NS

_F32 = jnp.float32
_BF = jnp.bfloat16


def _ln(o, s, b):
    mu = jnp.mean(o, axis=-1, keepdims=True)
    d = o - mu
    var = jnp.mean(d * d, axis=-1, keepdims=True)
    return d * lax.rsqrt(var + 1e-5) * s + b


def _stats_body(x_ref, o_ref):
    x = x_ref[...]
    s = jnp.sum(x, axis=0)
    q = jnp.sum(x * x, axis=0)
    o_ref[...] = jnp.stack([s, q])[None]


def _col_stats(x, block):
    n, dp = x.shape
    grid = n // block
    out = pl.pallas_call(
        _stats_body,
        grid=(grid,),
        in_specs=[pl.BlockSpec((block, dp), lambda i: (i, 0))],
        out_specs=pl.BlockSpec((1, 2, dp), lambda i: (i, 0, 0)),
        out_shape=jax.ShapeDtypeStruct((grid, 2, dp), _F32),
    )(x)
    return jnp.sum(out, axis=0)


def _enc_body(x_ref, w0_ref, b0_ref, w1_ref, b1_ref, s_ref, t_ref, o_ref):
    h = jnp.dot(x_ref[...], w0_ref[...], preferred_element_type=_F32) + b0_ref[...]
    h = jnp.maximum(h, 0.0)
    o = jnp.dot(h, w1_ref[...], preferred_element_type=_F32) + b1_ref[...]
    o_ref[...] = _ln(o, s_ref[...], t_ref[...])


def _enc(x, w0, b0, w1, b1, lns, lnb, block):
    n, dp = x.shape
    grid = n // block
    wspec = [
        pl.BlockSpec((dp, _D), lambda i: (0, 0)),
        pl.BlockSpec((1, _D), lambda i: (0, 0)),
        pl.BlockSpec((_D, _D), lambda i: (0, 0)),
        pl.BlockSpec((1, _D), lambda i: (0, 0)),
        pl.BlockSpec((1, _D), lambda i: (0, 0)),
        pl.BlockSpec((1, _D), lambda i: (0, 0)),
    ]
    return pl.pallas_call(
        _enc_body,
        grid=(grid,),
        in_specs=[pl.BlockSpec((block, dp), lambda i: (i, 0))] + wspec,
        out_specs=pl.BlockSpec((block, _D), lambda i: (i, 0)),
        out_shape=jax.ShapeDtypeStruct((n, _D), _F32),
    )(x, w0, b0, w1, b1, lns, lnb)


def _prep_body(x_ref, ws_ref, wr_ref, ps_ref, pr_ref):
    x = x_ref[...]
    ps_ref[...] = jnp.dot(x, ws_ref[...], preferred_element_type=_F32)
    pr_ref[...] = jnp.dot(x, wr_ref[...], preferred_element_type=_F32)


def _prep(nodes, ws, wr, block=2000):
    grid = _N // block
    return pl.pallas_call(
        _prep_body,
        grid=(grid,),
        in_specs=[pl.BlockSpec((block, _D), lambda i: (i, 0)),
                  pl.BlockSpec((_D, _D), lambda i: (0, 0)),
                  pl.BlockSpec((_D, _D), lambda i: (0, 0))],
        out_specs=[pl.BlockSpec((block, _D), lambda i: (i, 0)),
                   pl.BlockSpec((block, _D), lambda i: (i, 0))],
        out_shape=[jax.ShapeDtypeStruct((_N, _D), _F32),
                   jax.ShapeDtypeStruct((_N, _D), _F32)],
    )(nodes, ws, wr)


def _edge_body(gs_ref, gr_ref, e_ref, we_ref, b0_ref, w1_ref, b1_ref,
               s_ref, t_ref, o_ref):
    e = e_ref[...]
    pre = jnp.dot(e, we_ref[...], preferred_element_type=_F32)
    pre = pre + gs_ref[...] + gr_ref[...] + b0_ref[...]
    h = jnp.maximum(pre, 0.0)
    o = jnp.dot(h, w1_ref[...], preferred_element_type=_F32) + b1_ref[...]
    o_ref[...] = e + _ln(o, s_ref[...], t_ref[...])


def _edge_mlp(gs, gr, edges, we, b0, w1, b1, lns, lnb, block=2000):
    grid = _E // block
    dspec = pl.BlockSpec((block, _D), lambda i: (i, 0))
    gspec = pl.BlockSpec((block, _D), lambda i: (i, 0))
    wspec = [
        pl.BlockSpec((_D, _D), lambda i: (0, 0)),
        pl.BlockSpec((1, _D), lambda i: (0, 0)),
        pl.BlockSpec((_D, _D), lambda i: (0, 0)),
        pl.BlockSpec((1, _D), lambda i: (0, 0)),
        pl.BlockSpec((1, _D), lambda i: (0, 0)),
        pl.BlockSpec((1, _D), lambda i: (0, 0)),
    ]
    return pl.pallas_call(
        _edge_body,
        grid=(grid,),
        in_specs=[gspec, gspec, dspec] + wspec,
        out_specs=dspec,
        out_shape=jax.ShapeDtypeStruct((_E, _D), _F32),
    )(gs, gr, edges, we, b0, w1, b1, lns, lnb)


def _node_body(x_ref, a0_ref, a1_ref, wn_ref, wa_ref, b0_ref, w1_ref, b1_ref,
               s_ref, t_ref, o_ref):
    x = x_ref[...]
    a = a0_ref[...] + a1_ref[...]
    pre = (jnp.dot(x, wn_ref[...], preferred_element_type=_F32)
           + jnp.dot(a, wa_ref[...], preferred_element_type=_F32)
           + b0_ref[...])
    h = jnp.maximum(pre, 0.0)
    o = jnp.dot(h, w1_ref[...], preferred_element_type=_F32) + b1_ref[...]
    o_ref[...] = x + _ln(o, s_ref[...], t_ref[...])


def _node_mlp(nodes, a0, a1, wn, wa, b0, w1, b1, lns, lnb, block=2000):
    grid = _N // block
    dspec = pl.BlockSpec((block, _D), lambda i: (i, 0))
    wspec = [
        pl.BlockSpec((_D, _D), lambda i: (0, 0)),
        pl.BlockSpec((_D, _D), lambda i: (0, 0)),
        pl.BlockSpec((1, _D), lambda i: (0, 0)),
        pl.BlockSpec((_D, _D), lambda i: (0, 0)),
        pl.BlockSpec((1, _D), lambda i: (0, 0)),
        pl.BlockSpec((1, _D), lambda i: (0, 0)),
        pl.BlockSpec((1, _D), lambda i: (0, 0)),
    ]
    return pl.pallas_call(
        _node_body,
        grid=(grid,),
        in_specs=[dspec, dspec, dspec] + wspec,
        out_specs=dspec,
        out_shape=jax.ShapeDtypeStruct((_N, _D), _F32),
    )(nodes, a0, a1, wn, wa, b0, w1, b1, lns, lnb)


def _dec_body(x_ref, w0_ref, b0_ref, w1_ref, b1_ref, o_ref):
    h = jnp.dot(x_ref[...], w0_ref[...], preferred_element_type=_F32) + b0_ref[...]
    h = jnp.maximum(h, 0.0)
    o_ref[...] = jnp.dot(h, w1_ref[...], preferred_element_type=_F32) + b1_ref[...]


def _dec(nodes, w0, b0, w1p, b1p, block=2000):
    grid = _N // block
    dspec = pl.BlockSpec((block, _D), lambda i: (i, 0))
    wspec = [
        pl.BlockSpec((_D, _D), lambda i: (0, 0)),
        pl.BlockSpec((1, _D), lambda i: (0, 0)),
        pl.BlockSpec((_D, _D), lambda i: (0, 0)),
        pl.BlockSpec((1, _D), lambda i: (0, 0)),
    ]
    return pl.pallas_call(
        _dec_body,
        grid=(grid,),
        in_specs=[dspec] + wspec,
        out_specs=dspec,
        out_shape=jax.ShapeDtypeStruct((_N, _D), _F32),
    )(nodes, w0, b0, w1p, b1p)


def _sc_mesh():
    return plsc.VectorSubcoreMesh(core_axis_name="c", subcore_axis_name="s")


def _gather_call(ps, pr, s_idx, r_idx):
    @functools.partial(
        pl.kernel,
        mesh=_sc_mesh(),
        out_type=[jax.ShapeDtypeStruct((_E, _D), _F32),
                  jax.ShapeDtypeStruct((_E, _D), _F32)],
        scratch_types=[pltpu.VMEM((_CH,), jnp.int32),
                       pltpu.VMEM((_CH,), jnp.int32),
                       pltpu.VMEM((_CH, _D), _F32),
                       pltpu.VMEM((_CH, _D), _F32),
                       pltpu.SemaphoreType.DMA,
                       pltpu.SemaphoreType.DMA,
                       pltpu.SemaphoreType.DMA,
                       pltpu.SemaphoreType.DMA],
    )
    def k(ps_hbm, pr_hbm, s_hbm, r_hbm, gs_hbm, gr_hbm,
          sidx_v, ridx_v, srow_v, rrow_v, sis, sir, sgs, sgr):
        wid = lax.axis_index("s") * _NC + lax.axis_index("c")
        base = wid * _EPW

        def body(i, carry):
            off = base + i * _CH
            di_s = pltpu.async_copy(s_hbm.at[pl.ds(off, _CH)], sidx_v, sis)
            di_r = pltpu.async_copy(r_hbm.at[pl.ds(off, _CH)], ridx_v, sir)
            di_s.wait()
            dg_s = pltpu.async_copy(ps_hbm.at[sidx_v], srow_v, sgs)
            di_r.wait()
            dg_r = pltpu.async_copy(pr_hbm.at[ridx_v], rrow_v, sgr)
            dg_s.wait()
            pltpu.sync_copy(srow_v, gs_hbm.at[pl.ds(off, _CH)])
            dg_r.wait()
            pltpu.sync_copy(rrow_v, gr_hbm.at[pl.ds(off, _CH)])
            return carry

        lax.fori_loop(0, _NCH, body, 0)

    return k(ps, pr, s_idx, r_idx)


def _scatter_call(edges, r_idx, zacc):
    @functools.partial(
        pl.kernel,
        mesh=_sc_mesh(),
        out_type=jax.ShapeDtypeStruct((2 * _RPAD, _D), _F32),
        scratch_types=[pltpu.VMEM((_CHS,), jnp.int32),
                       pltpu.VMEM((_CHS,), jnp.int32),
                       pltpu.VMEM((_CHS, _D), _F32),
                       pltpu.VMEM((_CHS, _D), _F32),
                       pltpu.VMEM_SHARED((_RPAD, _D), _F32),
                       pltpu.SemaphoreType.DMA,
                       pltpu.SemaphoreType.DMA,
                       pltpu.SemaphoreType.DMA,
                       pltpu.SemaphoreType.DMA,
                       pltpu.SemaphoreType.DMA,
                       pltpu.SemaphoreType.DMA],
    )
    def k(e_hbm, r_hbm, z_hbm, out_hbm, ia_v, ib_v, ea_v, eb_v, acc_sh,
          sia, sib, sea, seb, saa, sab):
        c = lax.axis_index("c")
        s = lax.axis_index("s")
        wid = s * _NC + c
        rows0 = s * _RPT
        pltpu.sync_copy(z_hbm.at[pl.ds(rows0, _RPT)], acc_sh.at[pl.ds(rows0, _RPT)])
        plsc.subcore_barrier()
        base = wid * _EPW

        def pair(i, carry):
            offa = base + (2 * i) * _CHS
            offb = offa + _CHS
            dia = pltpu.async_copy(r_hbm.at[pl.ds(offa, _CHS)], ia_v, sia)
            dea = pltpu.async_copy(e_hbm.at[pl.ds(offa, _CHS)], ea_v, sea)
            dib = pltpu.async_copy(r_hbm.at[pl.ds(offb, _CHS)], ib_v, sib)
            deb = pltpu.async_copy(e_hbm.at[pl.ds(offb, _CHS)], eb_v, seb)
            dia.wait()
            dea.wait()
            daa = pltpu.async_copy(ea_v, acc_sh.at[ia_v], saa, add=True)
            dib.wait()
            deb.wait()
            dab = pltpu.async_copy(eb_v, acc_sh.at[ib_v], sab, add=True)
            daa.wait()
            dab.wait()
            return carry

        lax.fori_loop(0, _NCHS // 2, pair, 0)
        offt = base + (_NCHS - 1) * _CHS
        pltpu.sync_copy(r_hbm.at[pl.ds(offt, _CHS)], ia_v)
        pltpu.sync_copy(e_hbm.at[pl.ds(offt, _CHS)], ea_v)
        pltpu.sync_copy(ea_v, acc_sh.at[ia_v], add=True)
        plsc.subcore_barrier()
        pltpu.sync_copy(acc_sh.at[pl.ds(rows0, _RPT)],
                        out_hbm.at[pl.ds(c * _RPAD + rows0, _RPT)])

    return k(edges, r_idx, zacc)


def _fold_norm(sums, count, w0, b0, din):
    s, q = sums[0], sums[1]
    mean = s / count
    std = jnp.sqrt(q / count - mean * mean)
    std = jnp.maximum(std, 1e-8)
    dp = s.shape[0]
    w0p = jnp.pad(w0, ((0, dp - din), (0, 0)))
    w0f = w0p / std[:, None]
    b0f = b0 - (mean / std) @ w0p
    return w0f, b0f.reshape(1, _D)


def _r1(v):
    return v.reshape(1, -1)


def kernel(node_features, edge_features, params, senders, receivers):
    s_idx = senders.astype(jnp.int32)
    r_idx = receivers.astype(jnp.int32)
    nf = jnp.pad(node_features, ((0, 0), (0, 16 - node_features.shape[1])))
    ef = jnp.pad(edge_features, ((0, 0), (0, 8 - edge_features.shape[1])))

    nstats = _col_stats(nf, block=1000)
    estats = _col_stats(ef, block=4000)

    pn = params['node_enc']
    w0f, b0f = _fold_norm(nstats, float(_N), pn['w0'], pn['b0'], node_features.shape[1])
    nodes = _enc(nf, w0f, b0f, pn['w1'], _r1(pn['b1']), _r1(pn['ln_s']),
                 _r1(pn['ln_b']), block=1000)

    pe = params['edge_enc']
    w0f, b0f = _fold_norm(estats, float(_E), pe['w0'], pe['b0'], edge_features.shape[1])
    edges = _enc(ef, w0f, b0f, pe['w1'], _r1(pe['b1']), _r1(pe['ln_s']),
                 _r1(pe['ln_b']), block=4000)

    zacc = jnp.zeros((_RPAD, _D), _F32)

    for blk in params['blocks']:
        be, bn = blk['edge'], blk['node']
        ws, wr, we = be['w0'][:_D], be['w0'][_D:2 * _D], be['w0'][2 * _D:]
        ps, pr = _prep(nodes, ws, wr)
        gs, gr = _gather_call(ps, pr, s_idx, r_idx)
        edges = _edge_mlp(gs, gr, edges, we, _r1(be['b0']), be['w1'],
                          _r1(be['b1']), _r1(be['ln_s']), _r1(be['ln_b']))
        scat = _scatter_call(edges, r_idx, zacc)
        a0 = lax.slice(scat, (0, 0), (_N, _D))
        a1 = lax.slice(scat, (_RPAD, 0), (_RPAD + _N, _D))
        wn, wa = bn['w0'][:_D], bn['w0'][_D:]
        nodes = _node_mlp(nodes, a0, a1, wn, wa, _r1(bn['b0']), bn['w1'],
                          _r1(bn['b1']), _r1(bn['ln_s']), _r1(bn['ln_b']))

    pd = params['decoder']
    w1p = jnp.pad(pd['w1'], ((0, 0), (0, _D - pd['w1'].shape[1])))
    b1p = jnp.pad(pd['b1'], (0, _D - pd['b1'].shape[0]))
    out = _dec(nodes, pd['w0'], _r1(pd['b0']), w1p, _r1(b1p))
    return out[:, :3]


# speedup vs baseline: 3.6979x; 1.0136x over previous
import functools

import jax
import jax.numpy as jnp
from jax import lax
from jax.experimental import pallas as pl
from jax.experimental.pallas import tpu as pltpu
from jax.experimental.pallas import tpu_sc as plsc

_N = 10000
_E = 320000
_D = 128
_NC, _NS = 2, 16
_NW = _NC * _NS
_EPW = _E // _NW
_CHS = 80
_NCHS = _EPW // _CHS
_RPAD = 10240
_RPT = _RPAD // _NS

_F32 = jnp.float32
_BF = jnp.bfloat16


def _ln(o, s, b):
    mu = jnp.mean(o, axis=-1, keepdims=True)
    d = o - mu
    var = jnp.mean(d * d, axis=-1, keepdims=True)
    return d * lax.rsqrt(var + 1e-5) * s + b


def _stats_body(x_ref, o_ref):
    x = x_ref[...]
    s = jnp.sum(x, axis=0)
    q = jnp.sum(x * x, axis=0)
    o_ref[...] = jnp.stack([s, q])[None]


def _col_stats(x, block):
    n, dp = x.shape
    grid = n // block
    out = pl.pallas_call(
        _stats_body,
        grid=(grid,),
        in_specs=[pl.BlockSpec((block, dp), lambda i: (i, 0))],
        out_specs=pl.BlockSpec((1, 2, dp), lambda i: (i, 0, 0)),
        out_shape=jax.ShapeDtypeStruct((grid, 2, dp), _F32),
    )(x)
    return jnp.sum(out, axis=0)


def _enc_body(x_ref, w0_ref, b0_ref, w1_ref, b1_ref, s_ref, t_ref, o_ref):
    h = jnp.dot(x_ref[...], w0_ref[...], preferred_element_type=_F32) + b0_ref[...]
    h = jnp.maximum(h, 0.0)
    o = jnp.dot(h, w1_ref[...], preferred_element_type=_F32) + b1_ref[...]
    o_ref[...] = _ln(o, s_ref[...], t_ref[...])


def _enc(x, w0, b0, w1, b1, lns, lnb, block):
    n, dp = x.shape
    grid = n // block
    wspec = [
        pl.BlockSpec((dp, _D), lambda i: (0, 0)),
        pl.BlockSpec((1, _D), lambda i: (0, 0)),
        pl.BlockSpec((_D, _D), lambda i: (0, 0)),
        pl.BlockSpec((1, _D), lambda i: (0, 0)),
        pl.BlockSpec((1, _D), lambda i: (0, 0)),
        pl.BlockSpec((1, _D), lambda i: (0, 0)),
    ]
    return pl.pallas_call(
        _enc_body,
        grid=(grid,),
        in_specs=[pl.BlockSpec((block, dp), lambda i: (i, 0))] + wspec,
        out_specs=pl.BlockSpec((block, _D), lambda i: (i, 0)),
        out_shape=jax.ShapeDtypeStruct((n, _D), _F32),
    )(x, w0, b0, w1, b1, lns, lnb)


_U32 = jnp.uint32
_DH = _D // 2


def _pack128(x):
    a = x[:, :_DH]
    b = x[:, _DH:]
    ua = lax.shift_right_logical(
        lax.bitcast_convert_type(a.astype(_BF).astype(_F32), _U32), _U32(16))
    ub = lax.bitwise_and(
        lax.bitcast_convert_type(b.astype(_BF).astype(_F32), _U32), _U32(0xFFFF0000))
    return lax.bitcast_convert_type(lax.bitwise_or(ua, ub), _F32)


def _unpack128(g):
    w = lax.bitcast_convert_type(g, _U32)
    a = lax.bitcast_convert_type(lax.shift_left(w, _U32(16)), _F32)
    b = lax.bitcast_convert_type(lax.bitwise_and(w, _U32(0xFFFF0000)), _F32)
    return jnp.concatenate([a, b], axis=1)


def _prep_body(x_ref, ws_ref, wr_ref, t_ref):
    x = x_ref[...]
    ps = jnp.dot(x, ws_ref[...], preferred_element_type=_F32)
    pr = jnp.dot(x, wr_ref[...], preferred_element_type=_F32)
    t_ref[...] = jnp.concatenate([_pack128(ps), _pack128(pr)], axis=1)


def _prep(nodes, ws, wr, block=2000):
    grid = _N // block
    return pl.pallas_call(
        _prep_body,
        grid=(grid,),
        in_specs=[pl.BlockSpec((block, _D), lambda i: (i, 0)),
                  pl.BlockSpec((_D, _D), lambda i: (0, 0)),
                  pl.BlockSpec((_D, _D), lambda i: (0, 0))],
        out_specs=pl.BlockSpec((block, _D), lambda i: (i, 0)),
        out_shape=jax.ShapeDtypeStruct((_N, _D), _F32),
    )(nodes, ws, wr)


def _edge_body(g_ref, e_ref, we_ref, b0_ref, w1_ref, b1_ref,
               s_ref, t_ref, o_ref):
    e = e_ref[...]
    g = g_ref[...]
    pre = jnp.dot(e, we_ref[...], preferred_element_type=_F32)
    pre = pre + _unpack128(g[:, :_DH]) + _unpack128(g[:, _DH:]) + b0_ref[...]
    h = jnp.maximum(pre, 0.0)
    o = jnp.dot(h, w1_ref[...], preferred_element_type=_F32) + b1_ref[...]
    o_ref[...] = e + _ln(o, s_ref[...], t_ref[...])


def _edge_mlp(g, edges, we, b0, w1, b1, lns, lnb, block=2000):
    grid = _E // block
    dspec = pl.BlockSpec((block, _D), lambda i: (i, 0))
    wspec = [
        pl.BlockSpec((_D, _D), lambda i: (0, 0)),
        pl.BlockSpec((1, _D), lambda i: (0, 0)),
        pl.BlockSpec((_D, _D), lambda i: (0, 0)),
        pl.BlockSpec((1, _D), lambda i: (0, 0)),
        pl.BlockSpec((1, _D), lambda i: (0, 0)),
        pl.BlockSpec((1, _D), lambda i: (0, 0)),
    ]
    return pl.pallas_call(
        _edge_body,
        grid=(grid,),
        in_specs=[dspec, dspec] + wspec,
        out_specs=dspec,
        out_shape=jax.ShapeDtypeStruct((_E, _D), _F32),
    )(g, edges, we, b0, w1, b1, lns, lnb)


def _node_body(x_ref, a0_ref, a1_ref, wn_ref, wa_ref, b0_ref, w1_ref, b1_ref,
               s_ref, t_ref, o_ref):
    x = x_ref[...]
    a = a0_ref[...] + a1_ref[...]
    pre = (jnp.dot(x, wn_ref[...], preferred_element_type=_F32)
           + jnp.dot(a, wa_ref[...], preferred_element_type=_F32)
           + b0_ref[...])
    h = jnp.maximum(pre, 0.0)
    o = jnp.dot(h, w1_ref[...], preferred_element_type=_F32) + b1_ref[...]
    o_ref[...] = x + _ln(o, s_ref[...], t_ref[...])


def _node_mlp(nodes, a0, a1, wn, wa, b0, w1, b1, lns, lnb, block=2000):
    grid = _N // block
    dspec = pl.BlockSpec((block, _D), lambda i: (i, 0))
    wspec = [
        pl.BlockSpec((_D, _D), lambda i: (0, 0)),
        pl.BlockSpec((_D, _D), lambda i: (0, 0)),
        pl.BlockSpec((1, _D), lambda i: (0, 0)),
        pl.BlockSpec((_D, _D), lambda i: (0, 0)),
        pl.BlockSpec((1, _D), lambda i: (0, 0)),
        pl.BlockSpec((1, _D), lambda i: (0, 0)),
        pl.BlockSpec((1, _D), lambda i: (0, 0)),
    ]
    return pl.pallas_call(
        _node_body,
        grid=(grid,),
        in_specs=[dspec, dspec, dspec] + wspec,
        out_specs=dspec,
        out_shape=jax.ShapeDtypeStruct((_N, _D), _F32),
    )(nodes, a0, a1, wn, wa, b0, w1, b1, lns, lnb)


def _dec_body(x_ref, w0_ref, b0_ref, w1_ref, b1_ref, o_ref):
    h = jnp.dot(x_ref[...], w0_ref[...], preferred_element_type=_F32) + b0_ref[...]
    h = jnp.maximum(h, 0.0)
    o_ref[...] = jnp.dot(h, w1_ref[...], preferred_element_type=_F32) + b1_ref[...]


def _dec(nodes, w0, b0, w1p, b1p, block=2000):
    grid = _N // block
    dspec = pl.BlockSpec((block, _D), lambda i: (i, 0))
    wspec = [
        pl.BlockSpec((_D, _D), lambda i: (0, 0)),
        pl.BlockSpec((1, _D), lambda i: (0, 0)),
        pl.BlockSpec((_D, _D), lambda i: (0, 0)),
        pl.BlockSpec((1, _D), lambda i: (0, 0)),
    ]
    return pl.pallas_call(
        _dec_body,
        grid=(grid,),
        in_specs=[dspec] + wspec,
        out_specs=dspec,
        out_shape=jax.ShapeDtypeStruct((_N, _D), _F32),
    )(nodes, w0, b0, w1p, b1p)


def _sc_mesh():
    return plsc.VectorSubcoreMesh(core_axis_name="c", subcore_axis_name="s")


def _gather_call(tbl, s_idx, r_idx):
    cbuf = [pltpu.VMEM((_CHS,), jnp.int32),
            pltpu.VMEM((_CHS,), jnp.int32),
            pltpu.VMEM((_CHS, _D), _F32),
            pltpu.VMEM((_CHS, _D), _F32),
            pltpu.VMEM((_CHS, _D), _F32)]

    @functools.partial(
        pl.kernel,
        mesh=_sc_mesh(),
        out_type=jax.ShapeDtypeStruct((_E, _D), _F32),
        scratch_types=cbuf + cbuf + [pltpu.SemaphoreType.DMA] * 10,
    )
    def k(t_hbm, s_hbm, r_hbm, g_hbm,
          sia_v, ria_v, sra_v, rra_v, ga_v,
          sib_v, rib_v, srb_v, rrb_v, gb_v,
          s1, s2, s3, s4, s5, s6, s7, s8, s9, s10):
        wid = lax.axis_index("s") * _NC + lax.axis_index("c")
        base = wid * _EPW

        def combine(srow_v, rrow_v, gout_v):
            def comb(j, carry):
                for q in range(_DH // 16):
                    gout_v[j, pl.ds(16 * q, 16)] = srow_v[j, pl.ds(16 * q, 16)]
                    gout_v[j, pl.ds(_DH + 16 * q, 16)] = (
                        rrow_v[j, pl.ds(_DH + 16 * q, 16)])
                return carry
            lax.fori_loop(0, _CHS, comb, 0)

        def chunk_start(off, sidx_v, ridx_v, semi_s, semi_r):
            di_s = pltpu.async_copy(s_hbm.at[pl.ds(off, _CHS)], sidx_v, semi_s)
            di_r = pltpu.async_copy(r_hbm.at[pl.ds(off, _CHS)], ridx_v, semi_r)
            return di_s, di_r

        def gather_start(descs, sidx_v, ridx_v, srow_v, rrow_v, semg_s, semg_r):
            descs[0].wait()
            dg_s = pltpu.async_copy(t_hbm.at[sidx_v], srow_v, semg_s)
            descs[1].wait()
            dg_r = pltpu.async_copy(t_hbm.at[ridx_v], rrow_v, semg_r)
            return dg_s, dg_r

        def body(i, carry):
            offa = base + (2 * i) * _CHS
            offb = offa + _CHS
            da = chunk_start(offa, sia_v, ria_v, s1, s2)
            db = chunk_start(offb, sib_v, rib_v, s3, s4)
            ga = gather_start(da, sia_v, ria_v, sra_v, rra_v, s5, s6)
            gb = gather_start(db, sib_v, rib_v, srb_v, rrb_v, s7, s8)
            ga[0].wait()
            ga[1].wait()
            combine(sra_v, rra_v, ga_v)
            dwa = pltpu.async_copy(ga_v, g_hbm.at[pl.ds(offa, _CHS)], s9)
            gb[0].wait()
            gb[1].wait()
            combine(srb_v, rrb_v, gb_v)
            dwb = pltpu.async_copy(gb_v, g_hbm.at[pl.ds(offb, _CHS)], s10)
            dwa.wait()
            dwb.wait()
            return carry

        lax.fori_loop(0, _NCHS // 2, body, 0)
        offt = base + (_NCHS - 1) * _CHS
        dt = chunk_start(offt, sia_v, ria_v, s1, s2)
        gt = gather_start(dt, sia_v, ria_v, sra_v, rra_v, s5, s6)
        gt[0].wait()
        gt[1].wait()
        combine(sra_v, rra_v, ga_v)
        pltpu.sync_copy(ga_v, g_hbm.at[pl.ds(offt, _CHS)])

    return k(tbl, s_idx, r_idx)


def _scatter_call(edges, r_idx, zacc):
    @functools.partial(
        pl.kernel,
        mesh=_sc_mesh(),
        out_type=jax.ShapeDtypeStruct((2 * _RPAD, _D), _F32),
        scratch_types=[pltpu.VMEM((_CHS,), jnp.int32),
                       pltpu.VMEM((_CHS,), jnp.int32),
                       pltpu.VMEM((_CHS, _D), _F32),
                       pltpu.VMEM((_CHS, _D), _F32),
                       pltpu.VMEM_SHARED((_RPAD, _D), _F32),
                       pltpu.SemaphoreType.DMA,
                       pltpu.SemaphoreType.DMA,
                       pltpu.SemaphoreType.DMA,
                       pltpu.SemaphoreType.DMA,
                       pltpu.SemaphoreType.DMA,
                       pltpu.SemaphoreType.DMA],
    )
    def k(e_hbm, r_hbm, z_hbm, out_hbm, ia_v, ib_v, ea_v, eb_v, acc_sh,
          sia, sib, sea, seb, saa, sab):
        c = lax.axis_index("c")
        s = lax.axis_index("s")
        wid = s * _NC + c
        rows0 = s * _RPT
        pltpu.sync_copy(z_hbm.at[pl.ds(rows0, _RPT)], acc_sh.at[pl.ds(rows0, _RPT)])
        plsc.subcore_barrier()
        base = wid * _EPW

        def pair(i, carry):
            offa = base + (2 * i) * _CHS
            offb = offa + _CHS
            dia = pltpu.async_copy(r_hbm.at[pl.ds(offa, _CHS)], ia_v, sia)
            dea = pltpu.async_copy(e_hbm.at[pl.ds(offa, _CHS)], ea_v, sea)
            dib = pltpu.async_copy(r_hbm.at[pl.ds(offb, _CHS)], ib_v, sib)
            deb = pltpu.async_copy(e_hbm.at[pl.ds(offb, _CHS)], eb_v, seb)
            dia.wait()
            dea.wait()
            daa = pltpu.async_copy(ea_v, acc_sh.at[ia_v], saa, add=True)
            dib.wait()
            deb.wait()
            dab = pltpu.async_copy(eb_v, acc_sh.at[ib_v], sab, add=True)
            daa.wait()
            dab.wait()
            return carry

        lax.fori_loop(0, _NCHS // 2, pair, 0)
        offt = base + (_NCHS - 1) * _CHS
        pltpu.sync_copy(r_hbm.at[pl.ds(offt, _CHS)], ia_v)
        pltpu.sync_copy(e_hbm.at[pl.ds(offt, _CHS)], ea_v)
        pltpu.sync_copy(ea_v, acc_sh.at[ia_v], add=True)
        plsc.subcore_barrier()
        pltpu.sync_copy(acc_sh.at[pl.ds(rows0, _RPT)],
                        out_hbm.at[pl.ds(c * _RPAD + rows0, _RPT)])

    return k(edges, r_idx, zacc)


def _fold_norm(sums, count, w0, b0, din):
    s, q = sums[0], sums[1]
    mean = s / count
    std = jnp.sqrt(q / count - mean * mean)
    std = jnp.maximum(std, 1e-8)
    dp = s.shape[0]
    w0p = jnp.pad(w0, ((0, dp - din), (0, 0)))
    w0f = w0p / std[:, None]
    b0f = b0 - (mean / std) @ w0p
    return w0f, b0f.reshape(1, _D)


def _r1(v):
    return v.reshape(1, -1)


def kernel(node_features, edge_features, params, senders, receivers):
    s_idx = senders.astype(jnp.int32)
    r_idx = receivers.astype(jnp.int32)
    nf = jnp.pad(node_features, ((0, 0), (0, 16 - node_features.shape[1])))
    ef = jnp.pad(edge_features, ((0, 0), (0, 8 - edge_features.shape[1])))

    nstats = _col_stats(nf, block=1000)
    estats = _col_stats(ef, block=4000)

    pn = params['node_enc']
    w0f, b0f = _fold_norm(nstats, float(_N), pn['w0'], pn['b0'], node_features.shape[1])
    nodes = _enc(nf, w0f, b0f, pn['w1'], _r1(pn['b1']), _r1(pn['ln_s']),
                 _r1(pn['ln_b']), block=1000)

    pe = params['edge_enc']
    w0f, b0f = _fold_norm(estats, float(_E), pe['w0'], pe['b0'], edge_features.shape[1])
    edges = _enc(ef, w0f, b0f, pe['w1'], _r1(pe['b1']), _r1(pe['ln_s']),
                 _r1(pe['ln_b']), block=4000)

    zacc = jnp.zeros((_RPAD, _D), _F32)

    for blk in params['blocks']:
        be, bn = blk['edge'], blk['node']
        ws, wr, we = be['w0'][:_D], be['w0'][_D:2 * _D], be['w0'][2 * _D:]
        tbl = _prep(nodes, ws, wr)
        g = _gather_call(tbl, s_idx, r_idx)
        edges = _edge_mlp(g, edges, we, _r1(be['b0']), be['w1'],
                          _r1(be['b1']), _r1(be['ln_s']), _r1(be['ln_b']))
        scat = _scatter_call(edges, r_idx, zacc)
        a0 = lax.slice(scat, (0, 0), (_N, _D))
        a1 = lax.slice(scat, (_RPAD, 0), (_RPAD + _N, _D))
        wn, wa = bn['w0'][:_D], bn['w0'][_D:]
        nodes = _node_mlp(nodes, a0, a1, wn, wa, _r1(bn['b0']), bn['w1'],
                          _r1(bn['b1']), _r1(bn['ln_s']), _r1(bn['ln_b']))

    pd = params['decoder']
    w1p = jnp.pad(pd['w1'], ((0, 0), (0, _D - pd['w1'].shape[1])))
    b1p = jnp.pad(pd['b1'], (0, _D - pd['b1'].shape[0]))
    out = _dec(nodes, pd['w0'], _r1(pd['b0']), w1p, _r1(b1p))
    return out[:, :3]


# speedup vs baseline: 4.0279x; 1.0892x over previous
import functools

import jax
import jax.numpy as jnp
from jax import lax
from jax.experimental import pallas as pl
from jax.experimental.pallas import tpu as pltpu
from jax.experimental.pallas import tpu_sc as plsc

_N = 10000
_E = 320000
_D = 128
_NC, _NS = 2, 16
_NW = _NC * _NS
_EPW = _E // _NW
_CHS = 80
_NCHS = _EPW // _CHS
_CHG = 200
_NCHG = _EPW // _CHG
_RPAD = 10240
_RPT = _RPAD // _NS

_F32 = jnp.float32
_BF = jnp.bfloat16


def _ln(o, s, b):
    mu = jnp.mean(o, axis=-1, keepdims=True)
    d = o - mu
    var = jnp.mean(d * d, axis=-1, keepdims=True)
    return d * lax.rsqrt(var + 1e-5) * s + b


def _stats_body(x_ref, o_ref):
    x = x_ref[...]
    s = jnp.sum(x, axis=0)
    q = jnp.sum(x * x, axis=0)
    o_ref[...] = jnp.stack([s, q])[None]


def _col_stats(x, block):
    n, dp = x.shape
    grid = n // block
    out = pl.pallas_call(
        _stats_body,
        grid=(grid,),
        in_specs=[pl.BlockSpec((block, dp), lambda i: (i, 0))],
        out_specs=pl.BlockSpec((1, 2, dp), lambda i: (i, 0, 0)),
        out_shape=jax.ShapeDtypeStruct((grid, 2, dp), _F32),
    )(x)
    return jnp.sum(out, axis=0)


def _enc_body(x_ref, w0_ref, b0_ref, w1_ref, b1_ref, s_ref, t_ref, o_ref):
    h = jnp.dot(x_ref[...], w0_ref[...], preferred_element_type=_F32) + b0_ref[...]
    h = jnp.maximum(h, 0.0)
    o = jnp.dot(h, w1_ref[...], preferred_element_type=_F32) + b1_ref[...]
    o_ref[...] = _ln(o, s_ref[...], t_ref[...])


def _enc(x, w0, b0, w1, b1, lns, lnb, block):
    n, dp = x.shape
    grid = n // block
    wspec = [
        pl.BlockSpec((dp, _D), lambda i: (0, 0)),
        pl.BlockSpec((1, _D), lambda i: (0, 0)),
        pl.BlockSpec((_D, _D), lambda i: (0, 0)),
        pl.BlockSpec((1, _D), lambda i: (0, 0)),
        pl.BlockSpec((1, _D), lambda i: (0, 0)),
        pl.BlockSpec((1, _D), lambda i: (0, 0)),
    ]
    return pl.pallas_call(
        _enc_body,
        grid=(grid,),
        in_specs=[pl.BlockSpec((block, dp), lambda i: (i, 0))] + wspec,
        out_specs=pl.BlockSpec((block, _D), lambda i: (i, 0)),
        out_shape=jax.ShapeDtypeStruct((n, _D), _F32),
    )(x, w0, b0, w1, b1, lns, lnb)


_U32 = jnp.uint32
_DH = _D // 2


def _pack128(x):
    a = x[:, :_DH]
    b = x[:, _DH:]
    ua = lax.shift_right_logical(
        lax.bitcast_convert_type(a.astype(_BF).astype(_F32), _U32), _U32(16))
    ub = lax.bitwise_and(
        lax.bitcast_convert_type(b.astype(_BF).astype(_F32), _U32), _U32(0xFFFF0000))
    return lax.bitcast_convert_type(lax.bitwise_or(ua, ub), _F32)


def _unpack128(g):
    w = lax.bitcast_convert_type(g, _U32)
    a = lax.bitcast_convert_type(lax.shift_left(w, _U32(16)), _F32)
    b = lax.bitcast_convert_type(lax.bitwise_and(w, _U32(0xFFFF0000)), _F32)
    return jnp.concatenate([a, b], axis=1)


def _prep_body(x_ref, ws_ref, wr_ref, t_ref):
    x = x_ref[...]
    ps = jnp.dot(x, ws_ref[...], preferred_element_type=_F32)
    pr = jnp.dot(x, wr_ref[...], preferred_element_type=_F32)
    t_ref[...] = jnp.concatenate([_pack128(ps), _pack128(pr)], axis=1)


def _prep(nodes, ws, wr, block=2000):
    grid = _N // block
    return pl.pallas_call(
        _prep_body,
        grid=(grid,),
        in_specs=[pl.BlockSpec((block, _D), lambda i: (i, 0)),
                  pl.BlockSpec((_D, _D), lambda i: (0, 0)),
                  pl.BlockSpec((_D, _D), lambda i: (0, 0))],
        out_specs=pl.BlockSpec((block, _D), lambda i: (i, 0)),
        out_shape=jax.ShapeDtypeStruct((_N, _D), _F32),
    )(nodes, ws, wr)


def _edge_body(g_ref, e_ref, we_ref, b0_ref, w1_ref, b1_ref,
               s_ref, t_ref, o_ref):
    e = e_ref[...]
    g = g_ref[...]
    pre = jnp.dot(e, we_ref[...], preferred_element_type=_F32)
    pre = pre + _unpack128(g[:, :_DH]) + _unpack128(g[:, _DH:]) + b0_ref[...]
    h = jnp.maximum(pre, 0.0)
    o = jnp.dot(h, w1_ref[...], preferred_element_type=_F32) + b1_ref[...]
    o_ref[...] = e + _ln(o, s_ref[...], t_ref[...])


def _edge_mlp(g, edges, we, b0, w1, b1, lns, lnb, block=2000):
    grid = _E // block
    dspec = pl.BlockSpec((block, _D), lambda i: (i, 0))
    wspec = [
        pl.BlockSpec((_D, _D), lambda i: (0, 0)),
        pl.BlockSpec((1, _D), lambda i: (0, 0)),
        pl.BlockSpec((_D, _D), lambda i: (0, 0)),
        pl.BlockSpec((1, _D), lambda i: (0, 0)),
        pl.BlockSpec((1, _D), lambda i: (0, 0)),
        pl.BlockSpec((1, _D), lambda i: (0, 0)),
    ]
    return pl.pallas_call(
        _edge_body,
        grid=(grid,),
        in_specs=[dspec, dspec] + wspec,
        out_specs=dspec,
        out_shape=jax.ShapeDtypeStruct((_E, _D), _F32),
    )(g, edges, we, b0, w1, b1, lns, lnb)


def _node_body(x_ref, a0_ref, a1_ref, wn_ref, wa_ref, b0_ref, w1_ref, b1_ref,
               s_ref, t_ref, o_ref):
    x = x_ref[...]
    a = a0_ref[...] + a1_ref[...]
    pre = (jnp.dot(x, wn_ref[...], preferred_element_type=_F32)
           + jnp.dot(a, wa_ref[...], preferred_element_type=_F32)
           + b0_ref[...])
    h = jnp.maximum(pre, 0.0)
    o = jnp.dot(h, w1_ref[...], preferred_element_type=_F32) + b1_ref[...]
    o_ref[...] = x + _ln(o, s_ref[...], t_ref[...])


def _node_mlp(nodes, a0, a1, wn, wa, b0, w1, b1, lns, lnb, block=2000):
    grid = _N // block
    dspec = pl.BlockSpec((block, _D), lambda i: (i, 0))
    wspec = [
        pl.BlockSpec((_D, _D), lambda i: (0, 0)),
        pl.BlockSpec((_D, _D), lambda i: (0, 0)),
        pl.BlockSpec((1, _D), lambda i: (0, 0)),
        pl.BlockSpec((_D, _D), lambda i: (0, 0)),
        pl.BlockSpec((1, _D), lambda i: (0, 0)),
        pl.BlockSpec((1, _D), lambda i: (0, 0)),
        pl.BlockSpec((1, _D), lambda i: (0, 0)),
    ]
    return pl.pallas_call(
        _node_body,
        grid=(grid,),
        in_specs=[dspec, dspec, dspec] + wspec,
        out_specs=dspec,
        out_shape=jax.ShapeDtypeStruct((_N, _D), _F32),
    )(nodes, a0, a1, wn, wa, b0, w1, b1, lns, lnb)


def _dec_body(x_ref, w0_ref, b0_ref, w1_ref, b1_ref, o_ref):
    h = jnp.dot(x_ref[...], w0_ref[...], preferred_element_type=_F32) + b0_ref[...]
    h = jnp.maximum(h, 0.0)
    o_ref[...] = jnp.dot(h, w1_ref[...], preferred_element_type=_F32) + b1_ref[...]


def _dec(nodes, w0, b0, w1p, b1p, block=2000):
    grid = _N // block
    dspec = pl.BlockSpec((block, _D), lambda i: (i, 0))
    wspec = [
        pl.BlockSpec((_D, _D), lambda i: (0, 0)),
        pl.BlockSpec((1, _D), lambda i: (0, 0)),
        pl.BlockSpec((_D, _D), lambda i: (0, 0)),
        pl.BlockSpec((1, _D), lambda i: (0, 0)),
    ]
    return pl.pallas_call(
        _dec_body,
        grid=(grid,),
        in_specs=[dspec] + wspec,
        out_specs=dspec,
        out_shape=jax.ShapeDtypeStruct((_N, _D), _F32),
    )(nodes, w0, b0, w1p, b1p)


def _sc_mesh():
    return plsc.VectorSubcoreMesh(core_axis_name="c", subcore_axis_name="s")


def _gather_call(tbl, s_idx, r_idx):
    cbuf = [pltpu.VMEM((_CHG,), jnp.int32),
            pltpu.VMEM((_CHG,), jnp.int32),
            pltpu.VMEM((_CHG, _D), _F32),
            pltpu.VMEM((_CHG, _D), _F32)]

    @functools.partial(
        pl.kernel,
        mesh=_sc_mesh(),
        out_type=jax.ShapeDtypeStruct((_E, _D), _F32),
        scratch_types=cbuf + cbuf + [pltpu.SemaphoreType.DMA] * 10,
    )
    def k(t_hbm, s_hbm, r_hbm, g_hbm,
          sia_v, ria_v, sra_v, rra_v,
          sib_v, rib_v, srb_v, rrb_v,
          s1, s2, s3, s4, s5, s6, s7, s8, s9, s10):
        wid = lax.axis_index("s") * _NC + lax.axis_index("c")
        base = wid * _EPW

        def combine(srow_v, rrow_v):
            def comb(j, carry):
                for q in range(_DH // 16):
                    srow_v[j, pl.ds(_DH + 16 * q, 16)] = (
                        rrow_v[j, pl.ds(_DH + 16 * q, 16)])
                return carry
            lax.fori_loop(0, _CHG, comb, 0)

        def chunk_start(off, sidx_v, ridx_v, semi_s, semi_r):
            di_s = pltpu.async_copy(s_hbm.at[pl.ds(off, _CHG)], sidx_v, semi_s)
            di_r = pltpu.async_copy(r_hbm.at[pl.ds(off, _CHG)], ridx_v, semi_r)
            return di_s, di_r

        def gather_start(descs, sidx_v, ridx_v, srow_v, rrow_v, semg_s, semg_r):
            descs[0].wait()
            dg_s = pltpu.async_copy(t_hbm.at[sidx_v], srow_v, semg_s)
            descs[1].wait()
            dg_r = pltpu.async_copy(t_hbm.at[ridx_v], rrow_v, semg_r)
            return dg_s, dg_r

        def body(i, carry):
            offa = base + (2 * i) * _CHG
            offb = offa + _CHG
            da = chunk_start(offa, sia_v, ria_v, s1, s2)
            db = chunk_start(offb, sib_v, rib_v, s3, s4)
            ga = gather_start(da, sia_v, ria_v, sra_v, rra_v, s5, s6)
            gb = gather_start(db, sib_v, rib_v, srb_v, rrb_v, s7, s8)
            ga[0].wait()
            ga[1].wait()
            combine(sra_v, rra_v)
            dwa = pltpu.async_copy(sra_v, g_hbm.at[pl.ds(offa, _CHG)], s9)
            gb[0].wait()
            gb[1].wait()
            combine(srb_v, rrb_v)
            dwb = pltpu.async_copy(srb_v, g_hbm.at[pl.ds(offb, _CHG)], s10)
            dwa.wait()
            dwb.wait()
            return carry

        lax.fori_loop(0, _NCHG // 2, body, 0)

    return k(tbl, s_idx, r_idx)


def _scatter_call(edges, r_idx, zacc):
    @functools.partial(
        pl.kernel,
        mesh=_sc_mesh(),
        out_type=jax.ShapeDtypeStruct((2 * _RPAD, _D), _F32),
        scratch_types=[pltpu.VMEM((4, _CHS), jnp.int32),
                       pltpu.VMEM((4 * _CHS, _D), _F32),
                       pltpu.VMEM_SHARED((_RPAD, _D), _F32)]
                      + [pltpu.SemaphoreType.DMA] * 12,
    )
    def k(e_hbm, r_hbm, z_hbm, out_hbm, idx_v, ev, acc_sh,
          si0, si1, si2, si3, se0, se1, se2, se3, sa0, sa1, sa2, sa3):
        c = lax.axis_index("c")
        s = lax.axis_index("s")
        wid = s * _NC + c
        rows0 = s * _RPT
        pltpu.sync_copy(z_hbm.at[pl.ds(rows0, _RPT)], acc_sh.at[pl.ds(rows0, _RPT)])
        plsc.subcore_barrier()
        base = wid * _EPW
        sis = [si0, si1, si2, si3]
        ses = [se0, se1, se2, se3]
        sas = [sa0, sa1, sa2, sa3]

        def quad(i, carry):
            off = base + (4 * i) * _CHS
            dl = []
            for q in range(4):
                oq = off + q * _CHS
                dl.append((
                    pltpu.async_copy(r_hbm.at[pl.ds(oq, _CHS)],
                                     idx_v.at[q], sis[q]),
                    pltpu.async_copy(e_hbm.at[pl.ds(oq, _CHS)],
                                     ev.at[pl.ds(q * _CHS, _CHS)], ses[q])))
            da = []
            for q in range(4):
                dl[q][0].wait()
                dl[q][1].wait()
                da.append(pltpu.async_copy(ev.at[pl.ds(q * _CHS, _CHS)],
                                           acc_sh.at[idx_v.at[q]], sas[q],
                                           add=True))
            for q in range(4):
                da[q].wait()
            return carry

        lax.fori_loop(0, _NCHS // 4, quad, 0)
        offt = base + (_NCHS - 1) * _CHS
        pltpu.sync_copy(r_hbm.at[pl.ds(offt, _CHS)], idx_v.at[0])
        pltpu.sync_copy(e_hbm.at[pl.ds(offt, _CHS)], ev.at[pl.ds(0, _CHS)])
        pltpu.sync_copy(ev.at[pl.ds(0, _CHS)], acc_sh.at[idx_v.at[0]], add=True)
        plsc.subcore_barrier()
        pltpu.sync_copy(acc_sh.at[pl.ds(rows0, _RPT)],
                        out_hbm.at[pl.ds(c * _RPAD + rows0, _RPT)])

    return k(edges, r_idx, zacc)


def _fold_norm(sums, count, w0, b0, din):
    s, q = sums[0], sums[1]
    mean = s / count
    std = jnp.sqrt(q / count - mean * mean)
    std = jnp.maximum(std, 1e-8)
    dp = s.shape[0]
    w0p = jnp.pad(w0, ((0, dp - din), (0, 0)))
    w0f = w0p / std[:, None]
    b0f = b0 - (mean / std) @ w0p
    return w0f, b0f.reshape(1, _D)


def _r1(v):
    return v.reshape(1, -1)


def kernel(node_features, edge_features, params, senders, receivers):
    s_idx = senders.astype(jnp.int32)
    r_idx = receivers.astype(jnp.int32)
    nf = jnp.pad(node_features, ((0, 0), (0, 16 - node_features.shape[1])))
    ef = jnp.pad(edge_features, ((0, 0), (0, 8 - edge_features.shape[1])))

    nstats = _col_stats(nf, block=1000)
    estats = _col_stats(ef, block=4000)

    pn = params['node_enc']
    w0f, b0f = _fold_norm(nstats, float(_N), pn['w0'], pn['b0'], node_features.shape[1])
    nodes = _enc(nf, w0f, b0f, pn['w1'], _r1(pn['b1']), _r1(pn['ln_s']),
                 _r1(pn['ln_b']), block=1000)

    pe = params['edge_enc']
    w0f, b0f = _fold_norm(estats, float(_E), pe['w0'], pe['b0'], edge_features.shape[1])
    edges = _enc(ef, w0f, b0f, pe['w1'], _r1(pe['b1']), _r1(pe['ln_s']),
                 _r1(pe['ln_b']), block=4000)

    zacc = jnp.zeros((_RPAD, _D), _F32)

    for blk in params['blocks']:
        be, bn = blk['edge'], blk['node']
        ws, wr, we = be['w0'][:_D], be['w0'][_D:2 * _D], be['w0'][2 * _D:]
        tbl = _prep(nodes, ws, wr)
        g = _gather_call(tbl, s_idx, r_idx)
        edges = _edge_mlp(g, edges, we, _r1(be['b0']), be['w1'],
                          _r1(be['b1']), _r1(be['ln_s']), _r1(be['ln_b']))
        scat = _scatter_call(edges, r_idx, zacc)
        a0 = lax.slice(scat, (0, 0), (_N, _D))
        a1 = lax.slice(scat, (_RPAD, 0), (_RPAD + _N, _D))
        wn, wa = bn['w0'][:_D], bn['w0'][_D:]
        nodes = _node_mlp(nodes, a0, a1, wn, wa, _r1(bn['b0']), bn['w1'],
                          _r1(bn['b1']), _r1(bn['ln_s']), _r1(bn['ln_b']))

    pd = params['decoder']
    w1p = jnp.pad(pd['w1'], ((0, 0), (0, _D - pd['w1'].shape[1])))
    b1p = jnp.pad(pd['b1'], (0, _D - pd['b1'].shape[0]))
    out = _dec(nodes, pd['w0'], _r1(pd['b0']), w1p, _r1(b1p))
    return out[:, :3]
